# baseline (speedup 1.0000x reference)
import functools

import jax
import jax.numpy as jnp
from jax import lax
from jax.experimental import pallas as pl
from jax.experimental.pallas import tpu as pltpu
from jax.experimental.pallas import tpu_sc as plsc


def _gather_kernel(rows_per_w, L, D, NC,
                   idx_hbm, table_hbm, out_hbm,
                   idx_v, rows_v, gsem0, gsem1, ssem0, ssem1):
    wid = lax.axis_index("s") * NC + lax.axis_index("c")
    base = wid * rows_per_w
    gsems = (gsem0, gsem1)
    ssems = (ssem0, ssem1)

    pltpu.sync_copy(idx_hbm.at[pl.ds(base, rows_per_w)], idx_v)

    def fire_gather(i, b):
        pltpu.async_copy(table_hbm.at[idx_v.at[i]], rows_v.at[b], gsems[b])

    def drain_gather(i, b):
        pltpu.make_async_copy(
            table_hbm.at[idx_v.at[i]], rows_v.at[b], gsems[b]).wait()

    def fire_scatter(i, b):
        pltpu.async_copy(rows_v.at[b], out_hbm.at[base + i], ssems[b])

    def drain_scatter(i, b):
        pltpu.make_async_copy(
            rows_v.at[b], out_hbm.at[base + i], ssems[b]).wait()

    fire_gather(0, 0)
    fire_gather(1, 1)
    drain_gather(0, 0)
    fire_scatter(0, 0)
    drain_scatter(0, 0)
    fire_gather(2, 0)
    drain_gather(1, 1)
    fire_scatter(1, 1)

    def pair(g, carry):
        i = 2 * g
        drain_scatter(i - 1, 1)
        fire_gather(i + 1, 1)
        drain_gather(i, 0)
        fire_scatter(i, 0)

        drain_scatter(i, 0)
        fire_gather(i + 2, 0)
        drain_gather(i + 1, 1)
        fire_scatter(i + 1, 1)
        return carry

    lax.fori_loop(1, rows_per_w // 2 - 1, pair, 0)

    i = rows_per_w - 2
    drain_scatter(i - 1, 1)
    fire_gather(i + 1, 1)
    drain_gather(i, 0)
    fire_scatter(i, 0)
    drain_scatter(i, 0)
    drain_gather(i + 1, 1)
    fire_scatter(i + 1, 1)
    drain_scatter(i + 1, 1)


def kernel(x, table):
    B, L = x.shape
    V, D = table.shape
    idx = x.astype(jnp.int32)

    info = plsc.get_sparse_core_info()
    NC, NS = info.num_cores, info.num_subcores
    NW = NC * NS
    assert B % NW == 0
    rows_per_w = B // NW
    assert rows_per_w % 2 == 0 and rows_per_w >= 4

    mesh = plsc.VectorSubcoreMesh(core_axis_name="c", subcore_axis_name="s")
    grid_kernel = pl.kernel(
        functools.partial(_gather_kernel, rows_per_w, L, D, NC),
        mesh=mesh,
        out_type=jax.ShapeDtypeStruct((B, L, D), jnp.float32),
        scratch_types=[
            pltpu.VMEM((rows_per_w, L), jnp.int32),
            pltpu.VMEM((2, L, D), jnp.float32),
            pltpu.SemaphoreType.DMA,
            pltpu.SemaphoreType.DMA,
            pltpu.SemaphoreType.DMA,
            pltpu.SemaphoreType.DMA,
        ],
        compiler_params=pltpu.CompilerParams(use_tc_tiling_on_sc=False),
    )
    return grid_kernel(idx, table)
